# baseline (speedup 1.0000x reference)
import functools

import jax
import jax.numpy as jnp
from jax import lax
from jax.experimental import pallas as pl
from jax.experimental.pallas import tpu as pltpu
from jax.experimental.pallas import tpu_sc as plsc

N = 100000
E = 100000
IN_C = 70
HID = 128
OUT_C = 70
ED = 4

NW = 32
GB = 128
EPW = 3200
EP = NW * EPW
NBG = EPW // GB

ROW_BLK = 2000
EBLK = 2048
CB = 1024

_f32 = jnp.float32


def _ln(t, g, b):
    m = jnp.mean(t, axis=-1, keepdims=True)
    v = jnp.mean(jnp.square(t - m), axis=-1, keepdims=True)
    return (t - m) * lax.rsqrt(v + 1e-5) * g + b


def _silu(t):
    return t * jax.nn.sigmoid(t)


def _dot(a, b):
    return jnp.dot(a, b, preferred_element_type=_f32)


def _full(shape):
    return pl.BlockSpec(shape, lambda i: (0, 0))


def _enc_body(x_ref, w_ref, b_ref, g_ref, bt_ref, o_ref):
    t = _dot(x_ref[...], w_ref[...]) + b_ref[...]
    o_ref[...] = _silu(_ln(t, g_ref[...], bt_ref[...]))


def _tc_encoder(x, w, b, g, bt):
    return pl.pallas_call(
        _enc_body,
        grid=(N // ROW_BLK,),
        in_specs=[
            pl.BlockSpec((ROW_BLK, IN_C), lambda i: (i, 0)),
            _full((IN_C, HID)),
            _full((1, HID)),
            _full((1, HID)),
            _full((1, HID)),
        ],
        out_specs=pl.BlockSpec((ROW_BLK, HID), lambda i: (i, 0)),
        out_shape=jax.ShapeDtypeStruct((N, HID), _f32),
    )(x, w, b.reshape(1, HID), g.reshape(1, HID), bt.reshape(1, HID))


def _send_body(g_ref, ea_ref, w1_ref, w2_ref, b_ref, gg_ref, bt_ref, o_ref):
    t = _dot(g_ref[...], w1_ref[...]) + _dot(ea_ref[...], w2_ref[...]) + b_ref[...]
    o_ref[...] = _silu(_ln(t, gg_ref[...], bt_ref[...]))


def _tc_sender(g, ea, lp):
    return pl.pallas_call(
        _send_body,
        grid=(EP // EBLK,),
        in_specs=[
            pl.BlockSpec((EBLK, HID), lambda i: (i, 0)),
            pl.BlockSpec((EBLK, ED), lambda i: (i, 0)),
            _full((HID, HID)),
            _full((ED, HID)),
            _full((1, HID)),
            _full((1, HID)),
            _full((1, HID)),
        ],
        out_specs=pl.BlockSpec((EBLK, HID), lambda i: (i, 0)),
        out_shape=jax.ShapeDtypeStruct((EP, HID), _f32),
    )(
        g, ea, lp["s_W"][:HID], lp["s_W"][HID:],
        lp["s_b"].reshape(1, HID), lp["s_g"].reshape(1, HID),
        lp["s_beta"].reshape(1, HID),
    )


def _cum_body(x_ref, l_ref, o_ref, carry_ref):
    @pl.when(pl.program_id(0) == 0)
    def _():
        carry_ref[...] = jnp.zeros_like(carry_ref)

    c = carry_ref[...]
    blk = x_ref[...]
    o_ref[...] = c + _dot(l_ref[...], blk)
    carry_ref[...] = c + jnp.sum(blk, axis=0, keepdims=True)


def _tc_exclusive_prefix(s_out, ltri):
    return pl.pallas_call(
        _cum_body,
        grid=(EP // CB,),
        in_specs=[
            pl.BlockSpec((CB, HID), lambda i: (i, 0)),
            _full((CB, CB)),
        ],
        out_specs=pl.BlockSpec((CB, HID), lambda i: (i, 0)),
        out_shape=jax.ShapeDtypeStruct((EP, HID), _f32),
        scratch_shapes=[pltpu.VMEM((1, HID), _f32)],
    )(s_out, ltri)


def _comb_body(h_ref, glu_ref, deg_ref, ea_ref, r1_ref, r2_ref,
               r3_ref, rb_ref, rg_ref, rbt_ref, n1g_ref, n1b_ref, o_ref):
    h = h_ref[...]
    agg = glu_ref[1] - glu_ref[0]
    invd = 1.0 / jnp.maximum(deg_ref[...], 1.0)
    t = (_dot(h, r1_ref[...]) + _dot(agg, r2_ref[...]) * invd
         + _dot(ea_ref[...], r3_ref[...]) + rb_ref[...])
    u = _silu(_ln(t, rg_ref[...], rbt_ref[...]))
    o_ref[...] = _ln(h + _silu(u), n1g_ref[...], n1b_ref[...])


def _tc_combined(h, glu, deg, ea, lp):
    rw = lp["r_W"]
    return pl.pallas_call(
        _comb_body,
        grid=(N // ROW_BLK,),
        in_specs=[
            pl.BlockSpec((ROW_BLK, HID), lambda i: (i, 0)),
            pl.BlockSpec((2, ROW_BLK, HID), lambda i: (0, i, 0)),
            pl.BlockSpec((ROW_BLK, 1), lambda i: (i, 0)),
            pl.BlockSpec((ROW_BLK, ED), lambda i: (i, 0)),
            _full((HID, HID)),
            _full((HID, HID)),
            _full((ED, HID)),
            _full((1, HID)),
            _full((1, HID)),
            _full((1, HID)),
            _full((1, HID)),
            _full((1, HID)),
        ],
        out_specs=pl.BlockSpec((ROW_BLK, HID), lambda i: (i, 0)),
        out_shape=jax.ShapeDtypeStruct((N, HID), _f32),
    )(
        h, glu, deg, ea, rw[:HID], rw[HID:2 * HID], rw[2 * HID:],
        lp["r_b"].reshape(1, HID), lp["r_g"].reshape(1, HID),
        lp["r_beta"].reshape(1, HID), lp["n1_g"].reshape(1, HID),
        lp["n1_beta"].reshape(1, HID),
    )


def _dec_body(h_ref, x_ref, w1_ref, b1_ref, g_ref, bt_ref, w2_ref, b2_ref,
              rw_ref, rb_ref, o_ref):
    d1 = _silu(_ln(_dot(h_ref[...], w1_ref[...]) + b1_ref[...],
                   g_ref[...], bt_ref[...]))
    o_ref[...] = (_dot(d1, w2_ref[...]) + b2_ref[...]
                  + _dot(x_ref[...], rw_ref[...]) + rb_ref[...])


def _tc_decoder(h, x, p):
    return pl.pallas_call(
        _dec_body,
        grid=(N // ROW_BLK,),
        in_specs=[
            pl.BlockSpec((ROW_BLK, HID), lambda i: (i, 0)),
            pl.BlockSpec((ROW_BLK, IN_C), lambda i: (i, 0)),
            _full((HID, HID)),
            _full((1, HID)),
            _full((1, HID)),
            _full((1, HID)),
            _full((HID, OUT_C)),
            _full((1, OUT_C)),
            _full((IN_C, OUT_C)),
            _full((1, OUT_C)),
        ],
        out_specs=pl.BlockSpec((ROW_BLK, OUT_C), lambda i: (i, 0)),
        out_shape=jax.ShapeDtypeStruct((N, OUT_C), _f32),
    )(
        h, x, p["dec_W1"], p["dec_b1"].reshape(1, HID),
        p["dec_g"].reshape(1, HID), p["dec_beta"].reshape(1, HID),
        p["dec_W2"], p["dec_b2"].reshape(1, OUT_C),
        p["res_W"], p["res_b"].reshape(1, OUT_C),
    )


@functools.cache
def _sc_gather_kernel(table_rows):
    del table_rows
    mesh = plsc.VectorSubcoreMesh(core_axis_name="c", subcore_axis_name="s")

    @functools.partial(
        pl.kernel,
        out_type=jax.ShapeDtypeStruct((EP, HID), _f32),
        mesh=mesh,
        scratch_types=[
            pltpu.VMEM((GB,), jnp.int32),
            pltpu.VMEM((GB, HID), _f32),
            pltpu.SemaphoreType.DMA,
        ],
    )
    def gk(tab_hbm, idx_hbm, out_hbm, idx_v, rows_v, sem):
        wid = lax.axis_index("s") * 2 + lax.axis_index("c")
        base = wid * EPW

        def body(k, _):
            off = base + k * GB
            pltpu.sync_copy(idx_hbm.at[pl.ds(off, GB)], idx_v)
            pltpu.async_copy(tab_hbm.at[idx_v], rows_v, sem).wait()
            pltpu.sync_copy(rows_v, out_hbm.at[pl.ds(off, GB)])
            return 0

        lax.fori_loop(0, NBG, body, 0)

    return gk


def _sc_gather(table, idx_pad):
    return _sc_gather_kernel(table.shape[0])(table, idx_pad)


@functools.cache
def _sc_gather2_kernel():
    mesh = plsc.VectorSubcoreMesh(core_axis_name="c", subcore_axis_name="s")

    @functools.partial(
        pl.kernel,
        out_type=jax.ShapeDtypeStruct((2, EP, HID), _f32),
        mesh=mesh,
        scratch_types=[
            pltpu.VMEM((GB,), jnp.int32),
            pltpu.VMEM((GB, HID), _f32),
            pltpu.SemaphoreType.DMA,
        ],
    )
    def gk(tab_hbm, idx_hbm, out_hbm, idx_v, rows_v, sem):
        wid = lax.axis_index("s") * 2 + lax.axis_index("c")
        base = wid * EPW

        for p in range(2):
            def body(k, _, p=p):
                off = base + k * GB
                pltpu.sync_copy(idx_hbm.at[p, pl.ds(off, GB)], idx_v)
                pltpu.async_copy(tab_hbm.at[idx_v], rows_v, sem).wait()
                pltpu.sync_copy(rows_v, out_hbm.at[p, pl.ds(off, GB)])
                return 0

            lax.fori_loop(0, NBG, body, 0)

    return gk


def _pad_idx(idx):
    return jnp.concatenate(
        [idx.astype(jnp.int32), jnp.zeros((EP - idx.shape[0],), jnp.int32)])


def kernel(x, edge_index, edge_attr, params):
    src = edge_index[0]
    dst = edge_index[1]

    order = jnp.argsort(dst)
    dst_s = dst[order]
    srcp = src[order]
    ea_s = edge_attr[order]

    srcp_p = _pad_idx(srcp)
    ea_sp = jnp.concatenate([ea_s, jnp.zeros((EP - E, ED), _f32)])

    bnd = jnp.searchsorted(
        dst_s, jnp.arange(N + 1, dtype=jnp.int32)).astype(jnp.int32)
    idx2 = jnp.stack([_pad_idx(bnd[:N]), _pad_idx(bnd[1:])])
    deg = (bnd[1:] - bnd[:N]).astype(_f32).reshape(N, 1)

    ltri = jnp.tril(jnp.ones((CB, CB), _f32), -1)

    h = _tc_encoder(x, params["enc_W"], params["enc_b"],
                    params["enc_g"], params["enc_beta"])

    for lp in params["layers"]:
        g = jnp.take(h, srcp_p, axis=0)
        s_out = _tc_sender(g, ea_sp, lp)
        t_pref = _tc_exclusive_prefix(s_out, ltri)
        glu = jnp.take(t_pref, idx2.reshape(-1), axis=0).reshape(2, EP, HID)
        h = _tc_combined(h, glu, deg, edge_attr, lp)

    return _tc_decoder(h, x, params)


# speedup vs baseline: 1.1467x; 1.1467x over previous
import functools

import jax
import jax.numpy as jnp
from jax import lax
from jax.experimental import pallas as pl
from jax.experimental.pallas import tpu as pltpu
from jax.experimental.pallas import tpu_sc as plsc

N = 100000
E = 100000
IN_C = 70
HID = 128
OUT_C = 70
ED = 4

NW = 32
GB = 128
EPW = 3200
EP = NW * EPW
NBG = EPW // GB

ROW_BLK = 2000
EBLK = 2048
CB = 1024

_f32 = jnp.float32


def _ln(t, g, b):
    m = jnp.mean(t, axis=-1, keepdims=True)
    v = jnp.mean(jnp.square(t - m), axis=-1, keepdims=True)
    return (t - m) * lax.rsqrt(v + 1e-5) * g + b


def _silu(t):
    return t * jax.nn.sigmoid(t)


def _dot(a, b):
    return jnp.dot(a, b, preferred_element_type=_f32)


def _full(shape):
    return pl.BlockSpec(shape, lambda i: (0, 0))


def _enc_body(x_ref, w_ref, b_ref, g_ref, bt_ref, o_ref):
    t = _dot(x_ref[...], w_ref[...]) + b_ref[...]
    o_ref[...] = _silu(_ln(t, g_ref[...], bt_ref[...]))


def _tc_encoder(x, w, b, g, bt):
    return pl.pallas_call(
        _enc_body,
        grid=(N // ROW_BLK,),
        in_specs=[
            pl.BlockSpec((ROW_BLK, IN_C), lambda i: (i, 0)),
            _full((IN_C, HID)),
            _full((1, HID)),
            _full((1, HID)),
            _full((1, HID)),
        ],
        out_specs=pl.BlockSpec((ROW_BLK, HID), lambda i: (i, 0)),
        out_shape=jax.ShapeDtypeStruct((N, HID), _f32),
    )(x, w, b.reshape(1, HID), g.reshape(1, HID), bt.reshape(1, HID))


def _send_body(g_ref, ea_ref, w1_ref, w2_ref, b_ref, gg_ref, bt_ref, o_ref):
    t = _dot(g_ref[...], w1_ref[...]) + _dot(ea_ref[...], w2_ref[...]) + b_ref[...]
    o_ref[...] = _silu(_ln(t, gg_ref[...], bt_ref[...]))


def _tc_sender(g, ea, lp):
    return pl.pallas_call(
        _send_body,
        grid=(EP // EBLK,),
        in_specs=[
            pl.BlockSpec((EBLK, HID), lambda i: (i, 0)),
            pl.BlockSpec((EBLK, ED), lambda i: (i, 0)),
            _full((HID, HID)),
            _full((ED, HID)),
            _full((1, HID)),
            _full((1, HID)),
            _full((1, HID)),
        ],
        out_specs=pl.BlockSpec((EBLK, HID), lambda i: (i, 0)),
        out_shape=jax.ShapeDtypeStruct((EP, HID), _f32),
    )(
        g, ea, lp["s_W"][:HID], lp["s_W"][HID:],
        lp["s_b"].reshape(1, HID), lp["s_g"].reshape(1, HID),
        lp["s_beta"].reshape(1, HID),
    )


def _cum_body(x_ref, l_ref, o_ref, carry_ref):
    @pl.when(pl.program_id(0) == 0)
    def _():
        carry_ref[...] = jnp.zeros_like(carry_ref)

    c = carry_ref[...]
    blk = x_ref[...]
    o_ref[...] = c + _dot(l_ref[...], blk)
    carry_ref[...] = c + jnp.sum(blk, axis=0, keepdims=True)


def _tc_exclusive_prefix(s_out, ltri):
    return pl.pallas_call(
        _cum_body,
        grid=(EP // CB,),
        in_specs=[
            pl.BlockSpec((CB, HID), lambda i: (i, 0)),
            _full((CB, CB)),
        ],
        out_specs=pl.BlockSpec((CB, HID), lambda i: (i, 0)),
        out_shape=jax.ShapeDtypeStruct((EP, HID), _f32),
        scratch_shapes=[pltpu.VMEM((1, HID), _f32)],
    )(s_out, ltri)


def _comb_body(h_ref, glu_ref, deg_ref, ea_ref, r1_ref, r2_ref,
               r3_ref, rb_ref, rg_ref, rbt_ref, n1g_ref, n1b_ref, o_ref):
    h = h_ref[...]
    agg = glu_ref[1] - glu_ref[0]
    invd = 1.0 / jnp.maximum(deg_ref[...], 1.0)
    t = (_dot(h, r1_ref[...]) + _dot(agg, r2_ref[...]) * invd
         + _dot(ea_ref[...], r3_ref[...]) + rb_ref[...])
    u = _silu(_ln(t, rg_ref[...], rbt_ref[...]))
    o_ref[...] = _ln(h + _silu(u), n1g_ref[...], n1b_ref[...])


def _tc_combined(h, glu, deg, ea, lp):
    rw = lp["r_W"]
    return pl.pallas_call(
        _comb_body,
        grid=(N // ROW_BLK,),
        in_specs=[
            pl.BlockSpec((ROW_BLK, HID), lambda i: (i, 0)),
            pl.BlockSpec((2, ROW_BLK, HID), lambda i: (0, i, 0)),
            pl.BlockSpec((ROW_BLK, 1), lambda i: (i, 0)),
            pl.BlockSpec((ROW_BLK, ED), lambda i: (i, 0)),
            _full((HID, HID)),
            _full((HID, HID)),
            _full((ED, HID)),
            _full((1, HID)),
            _full((1, HID)),
            _full((1, HID)),
            _full((1, HID)),
            _full((1, HID)),
        ],
        out_specs=pl.BlockSpec((ROW_BLK, HID), lambda i: (i, 0)),
        out_shape=jax.ShapeDtypeStruct((N, HID), _f32),
    )(
        h, glu, deg, ea, rw[:HID], rw[HID:2 * HID], rw[2 * HID:],
        lp["r_b"].reshape(1, HID), lp["r_g"].reshape(1, HID),
        lp["r_beta"].reshape(1, HID), lp["n1_g"].reshape(1, HID),
        lp["n1_beta"].reshape(1, HID),
    )


def _dec_body(h_ref, x_ref, w1_ref, b1_ref, g_ref, bt_ref, w2_ref, b2_ref,
              rw_ref, rb_ref, o_ref):
    d1 = _silu(_ln(_dot(h_ref[...], w1_ref[...]) + b1_ref[...],
                   g_ref[...], bt_ref[...]))
    o_ref[...] = (_dot(d1, w2_ref[...]) + b2_ref[...]
                  + _dot(x_ref[...], rw_ref[...]) + rb_ref[...])


def _tc_decoder(h, x, p):
    return pl.pallas_call(
        _dec_body,
        grid=(N // ROW_BLK,),
        in_specs=[
            pl.BlockSpec((ROW_BLK, HID), lambda i: (i, 0)),
            pl.BlockSpec((ROW_BLK, IN_C), lambda i: (i, 0)),
            _full((HID, HID)),
            _full((1, HID)),
            _full((1, HID)),
            _full((1, HID)),
            _full((HID, OUT_C)),
            _full((1, OUT_C)),
            _full((IN_C, OUT_C)),
            _full((1, OUT_C)),
        ],
        out_specs=pl.BlockSpec((ROW_BLK, OUT_C), lambda i: (i, 0)),
        out_shape=jax.ShapeDtypeStruct((N, OUT_C), _f32),
    )(
        h, x, p["dec_W1"], p["dec_b1"].reshape(1, HID),
        p["dec_g"].reshape(1, HID), p["dec_beta"].reshape(1, HID),
        p["dec_W2"], p["dec_b2"].reshape(1, OUT_C),
        p["res_W"], p["res_b"].reshape(1, OUT_C),
    )


@functools.cache
def _sc_gather_kernel(table_rows):
    del table_rows
    mesh = plsc.VectorSubcoreMesh(core_axis_name="c", subcore_axis_name="s")

    @functools.partial(
        pl.kernel,
        out_type=jax.ShapeDtypeStruct((EP, HID), _f32),
        mesh=mesh,
        scratch_types=[
            pltpu.VMEM((GB,), jnp.int32),
            pltpu.VMEM((GB, HID), _f32),
            pltpu.SemaphoreType.DMA,
        ],
    )
    def gk(tab_hbm, idx_hbm, out_hbm, idx_v, rows_v, sem):
        wid = lax.axis_index("s") * 2 + lax.axis_index("c")
        base = wid * EPW

        def body(k, _):
            off = base + k * GB
            pltpu.sync_copy(idx_hbm.at[pl.ds(off, GB)], idx_v)
            pltpu.async_copy(tab_hbm.at[idx_v], rows_v, sem).wait()
            pltpu.sync_copy(rows_v, out_hbm.at[pl.ds(off, GB)])
            return 0

        lax.fori_loop(0, NBG, body, 0)

    return gk


def _sc_gather(table, idx_pad):
    return _sc_gather_kernel(table.shape[0])(table, idx_pad)


@functools.cache
def _sc_gather2_kernel():
    mesh = plsc.VectorSubcoreMesh(core_axis_name="c", subcore_axis_name="s")

    @functools.partial(
        pl.kernel,
        out_type=jax.ShapeDtypeStruct((2, EP, HID), _f32),
        mesh=mesh,
        scratch_types=[
            pltpu.VMEM((GB,), jnp.int32),
            pltpu.VMEM((GB, HID), _f32),
            pltpu.SemaphoreType.DMA,
        ],
    )
    def gk(tab_hbm, idx_hbm, out_hbm, idx_v, rows_v, sem):
        wid = lax.axis_index("s") * 2 + lax.axis_index("c")
        base = wid * EPW

        for p in range(2):
            def body(k, _, p=p):
                off = base + k * GB
                pltpu.sync_copy(idx_hbm.at[p, pl.ds(off, GB)], idx_v)
                pltpu.async_copy(tab_hbm.at[idx_v], rows_v, sem).wait()
                pltpu.sync_copy(rows_v, out_hbm.at[p, pl.ds(off, GB)])
                return 0

            lax.fori_loop(0, NBG, body, 0)

    return gk


def _pad_idx(idx):
    return jnp.concatenate(
        [idx.astype(jnp.int32), jnp.zeros((EP - idx.shape[0],), jnp.int32)])


def kernel(x, edge_index, edge_attr, params):
    src = edge_index[0]
    dst = edge_index[1]

    order = jnp.argsort(dst)
    dst_s = dst[order]
    srcp = src[order]
    ea_s = edge_attr[order]

    srcp_p = _pad_idx(srcp)
    ea_sp = jnp.concatenate([ea_s, jnp.zeros((EP - E, ED), _f32)])

    bnd = jnp.searchsorted(
        dst_s, jnp.arange(N + 1, dtype=jnp.int32)).astype(jnp.int32)
    idx2 = jnp.stack([_pad_idx(bnd[:N]), _pad_idx(bnd[1:])])
    deg = (bnd[1:] - bnd[:N]).astype(_f32).reshape(N, 1)

    ltri = jnp.tril(jnp.ones((CB, CB), _f32), -1)

    h = _tc_encoder(x, params["enc_W"], params["enc_b"],
                    params["enc_g"], params["enc_beta"])

    for lp in params["layers"]:
        g = _sc_gather(h, srcp_p)
        s_out = _tc_sender(g, ea_sp, lp)
        t_pref = _tc_exclusive_prefix(s_out, ltri)
        glu = _sc_gather2_kernel()(t_pref, idx2)
        h = _tc_combined(h, glu, deg, edge_attr, lp)

    return _tc_decoder(h, x, params)
